# baseline (speedup 1.0000x reference)
import functools

import jax
import jax.numpy as jnp
from jax import lax
from jax.experimental import pallas as pl
from jax.experimental.pallas import tpu as pltpu
from jax.experimental.pallas import tpu_sc as plsc

N_PATCHES = 64
PATCH = 32
B = 8
C = 32
HL = WL = 48
HH = WH = 384
EPS = 1e-8
NV = HL * WL
ROWS_PER_PATCH = PATCH * 8
ROW_F32 = 128

NC, NS = 2, 16
NW = NC * NS
PPW = (B * N_PATCHES) // NW


def _sampler_body(att_ref, gum_ref, samp_ref, rows_ref):
    att = att_ref[...]
    scores = jnp.log(att + EPS) + gum_ref[...]
    pos = lax.broadcasted_iota(jnp.int32, (B, NV), 1)
    lane64 = lax.broadcasted_iota(jnp.int32, (B, N_PATCHES), 1)

    def step(i, carry):
        sc, fi, sa = carry
        m = jnp.max(sc, axis=1, keepdims=True)
        idx = jnp.min(jnp.where(sc == m, pos, NV), axis=1, keepdims=True)
        hit = pos == idx
        val = jnp.sum(jnp.where(hit, att, 0.0), axis=1, keepdims=True)
        onehot = lane64 == i
        fi = jnp.where(onehot, idx, fi)
        sa = jnp.where(onehot, val, sa)
        return jnp.where(hit, -jnp.inf, sc), fi, sa

    fi0 = jnp.zeros((B, N_PATCHES), jnp.int32)
    sa0 = jnp.zeros((B, N_PATCHES), jnp.float32)
    _, fi, sa = lax.fori_loop(0, N_PATCHES, step, (scores, fi0, sa0))
    samp_ref[...] = sa

    ys = fi // WL
    xs = fi - ys * WL
    top = jnp.clip(8 * ys - 12, 0, HH - PATCH)
    left = jnp.clip(8 * xs - 12, 0, WH - PATCH)
    bvec = lax.broadcasted_iota(jnp.int32, (B, N_PATCHES), 0)
    base = (bvec * HH + top) * 96 + left // 4
    k = lax.broadcasted_iota(jnp.int32, (B, N_PATCHES, ROWS_PER_PATCH), 2)
    rows_ref[...] = base[:, :, None] + (k // 8) * 96 + (k % 8)


def _run_sampler(att_flat, gumbel):
    return pl.pallas_call(
        _sampler_body,
        out_shape=(
            jax.ShapeDtypeStruct((B, N_PATCHES), jnp.float32),
            jax.ShapeDtypeStruct((B, N_PATCHES, ROWS_PER_PATCH), jnp.int32),
        ),
    )(att_flat, gumbel)


def _transpose_body(x_ref, o_ref):
    x = x_ref[0]
    t = x.reshape(C, 8 * WH).T
    o_ref[...] = t.reshape(1, 8, WH, C)


def _run_transpose(x_high):
    return pl.pallas_call(
        _transpose_body,
        grid=(B, HH // 8),
        in_specs=[pl.BlockSpec((1, C, 8, WH), lambda b, h: (b, 0, h, 0))],
        out_specs=pl.BlockSpec((1, 8, WH, C), lambda b, h: (b, h, 0, 0)),
        out_shape=jax.ShapeDtypeStruct((B, HH, WH, C), jnp.float32),
    )(x_high)


def _gather_body(table_hbm, rows_hbm, out_hbm, idx_v, buf_v, gsem):
    wid = lax.axis_index("s") * NC + lax.axis_index("c")
    for t in range(PPW):
        p = wid * PPW + t
        pltpu.sync_copy(rows_hbm.at[p], idx_v)
        cp0 = pltpu.async_copy(table_hbm.at[idx_v.at[0]], buf_v.at[pl.ds(0, 128)], gsem)
        cp1 = pltpu.async_copy(table_hbm.at[idx_v.at[1]], buf_v.at[pl.ds(128, 128)], gsem)
        cp0.wait()
        cp1.wait()
        pltpu.sync_copy(buf_v, out_hbm.at[pl.ds(p * ROWS_PER_PATCH, ROWS_PER_PATCH)])


def _run_gather(table, rows):
    mesh = plsc.VectorSubcoreMesh(core_axis_name="c", subcore_axis_name="s")
    k = functools.partial(
        pl.kernel,
        out_type=jax.ShapeDtypeStruct((B * N_PATCHES * ROWS_PER_PATCH, ROW_F32),
                                      jnp.float32),
        mesh=mesh,
        scratch_types=[
            pltpu.VMEM((2, 128), jnp.int32),
            pltpu.VMEM((ROWS_PER_PATCH, ROW_F32), jnp.float32),
            pltpu.SemaphoreType.DMA,
        ],
    )(_gather_body)
    return k(table, rows)


def kernel(x_low, x_high, attention, map_index):
    del x_low, map_index
    att_flat = attention.reshape(B, NV)
    u = jax.random.uniform(jax.random.key(42), (B, NV), minval=EPS, maxval=1.0)
    gumbel = -jnp.log(-jnp.log(u))

    samp, rows = _run_sampler(att_flat, gumbel)
    xh = _run_transpose(x_high)

    table = xh.reshape(B * HH * WH // 4, ROW_F32)
    rows2 = rows.reshape(B * N_PATCHES, 2, 128)
    out = _run_gather(table, rows2)
    patches = out.reshape(B, N_PATCHES, PATCH, PATCH, C)
    return patches, samp


# speedup vs baseline: 1.2598x; 1.2598x over previous
import functools

import numpy as np
import jax
import jax.numpy as jnp
from jax import lax
from jax.experimental import pallas as pl
from jax.experimental.pallas import tpu as pltpu
from jax.experimental.pallas import tpu_sc as plsc

N_PATCHES = 64
PATCH = 32
B = 8
C = 32
HL = WL = 48
HH = WH = 384
EPS = 1e-8
NV = HL * WL

CELL = 16
CPR = WH // CELL
N_CELLS = B * C * HH * CPR
CELLS_PER_PATCH = C * PATCH * 3
PATCH_F32 = PATCH * PATCH * C

NC, NS = 2, 16
NW = NC * NS
PPW = (B * N_PATCHES) // NW

_k = np.arange(CELLS_PER_PATCH)
_PATTERN = ((_k // (PATCH * 3)) * (HH * CPR)
            + ((_k % (PATCH * 3)) // 3) * CPR + (_k % 3)).astype(np.int32)


def _sampler_body(att_ref, gum_ref, samp_ref, meta_ref):
    att = att_ref[...]
    scores = jnp.log(att + EPS) + gum_ref[...]
    pos = lax.broadcasted_iota(jnp.int32, (B, NV), 1)
    lane64 = lax.broadcasted_iota(jnp.int32, (B, N_PATCHES), 1)

    def step(i, carry):
        sc, fi, sa = carry
        m = jnp.max(sc, axis=1, keepdims=True)
        idx = jnp.min(jnp.where(sc == m, pos, NV), axis=1, keepdims=True)
        hit = pos == idx
        val = jnp.sum(jnp.where(hit, att, 0.0), axis=1, keepdims=True)
        onehot = lane64 == i
        fi = jnp.where(onehot, idx, fi)
        sa = jnp.where(onehot, val, sa)
        return jnp.where(hit, -jnp.inf, sc), fi, sa

    fi0 = jnp.zeros((B, N_PATCHES), jnp.int32)
    sa0 = jnp.zeros((B, N_PATCHES), jnp.float32)
    _, fi, sa = lax.fori_loop(0, N_PATCHES, step, (scores, fi0, sa0))
    samp_ref[...] = sa

    ys = fi // WL
    xs = fi - ys * WL
    top = jnp.clip(8 * ys - 12, 0, HH - PATCH)
    left = jnp.clip(8 * xs - 12, 0, WH - PATCH)
    bvec = lax.broadcasted_iota(jnp.int32, (B, N_PATCHES), 0)
    base = bvec * (C * HH * CPR) + top * CPR + left // CELL
    sh = left % CELL
    meta = jnp.stack(
        [jnp.broadcast_to(base[:, :, None], (B, N_PATCHES, 16)),
         jnp.broadcast_to(sh[:, :, None], (B, N_PATCHES, 16))], axis=2)
    meta_ref[...] = meta


def _run_sampler(att_flat, gumbel):
    return pl.pallas_call(
        _sampler_body,
        out_shape=(
            jax.ShapeDtypeStruct((B, N_PATCHES), jnp.float32),
            jax.ShapeDtypeStruct((B, N_PATCHES, 2, 16), jnp.int32),
        ),
    )(att_flat, gumbel)


def _gather_body(src_hbm, pat_hbm, meta_hbm, out_hbm,
                 pat_v, mv, idx_v, in_v, out_v, gsem):
    wid = lax.axis_index("s") * NC + lax.axis_index("c")
    pltpu.sync_copy(pat_hbm, pat_v)
    lane = lax.iota(jnp.int32, 16)
    lane96 = lane * (PATCH * 3)

    def do_patch(t, _):
        p = wid * PPW + t
        pltpu.sync_copy(meta_hbm.at[p], mv)
        base_splat = mv[0, :]
        sh_splat = mv[1, :]
        clamp = jnp.full((16,), N_CELLS - 1, jnp.int32)

        def build_idx(k, _):
            vals = jnp.minimum(pat_v[pl.ds(k * 16, 16)] + base_splat, clamp)
            idx_v[k // 8, pl.ds((k % 8) * 16, 16)] = vals
            return 0

        lax.fori_loop(0, CELLS_PER_PATCH // 16, build_idx, 0)
        cps = [
            pltpu.async_copy(src_hbm.at[idx_v.at[kk]],
                             in_v.at[pl.ds(kk * 128, 128)], gsem)
            for kk in range(CELLS_PER_PATCH // 128)
        ]
        for cp in cps:
            cp.wait()

        def permute(ij, _):
            i = ij // PATCH
            j = ij - i * PATCH
            s2 = sh_splat + j
            q = s2 // CELL
            r = s2 - q * CELL
            row1 = lane96 + (i * 3) + q
            v1 = plsc.load_gather(in_v, [row1, r])
            v2 = plsc.load_gather(in_v, [row1 + 16 * PATCH * 3, r])
            off = i * (PATCH * C) + j * C
            out_v[pl.ds(off, 16)] = v1
            out_v[pl.ds(off + 16, 16)] = v2
            return 0

        lax.fori_loop(0, PATCH * PATCH, permute, 0)
        pltpu.sync_copy(out_v, out_hbm.at[p])
        return 0

    lax.fori_loop(0, PPW, do_patch, 0)


def _run_gather(src, pat, meta):
    mesh = plsc.VectorSubcoreMesh(core_axis_name="c", subcore_axis_name="s")
    k = functools.partial(
        pl.kernel,
        out_type=jax.ShapeDtypeStruct((B * N_PATCHES, PATCH_F32), jnp.float32),
        mesh=mesh,
        compiler_params=pltpu.CompilerParams(
            needs_layout_passes=False, use_tc_tiling_on_sc=False),
        scratch_types=[
            pltpu.VMEM((CELLS_PER_PATCH,), jnp.int32),
            pltpu.VMEM((2, 16), jnp.int32),
            pltpu.VMEM((CELLS_PER_PATCH // 128, 128), jnp.int32),
            pltpu.VMEM((CELLS_PER_PATCH, CELL), jnp.float32),
            pltpu.VMEM((PATCH_F32,), jnp.float32),
            pltpu.SemaphoreType.DMA,
        ],
    )(_gather_body)
    return k(src, pat, meta)


def kernel(x_low, x_high, attention, map_index):
    del x_low, map_index
    att_flat = attention.reshape(B, NV)
    u = jax.random.uniform(jax.random.key(42), (B, NV), minval=EPS, maxval=1.0)
    gumbel = -jnp.log(-jnp.log(u))

    samp, meta = _run_sampler(att_flat, gumbel)

    src = x_high.reshape(N_CELLS, CELL)
    out = _run_gather(src, jnp.asarray(_PATTERN),
                      meta.reshape(B * N_PATCHES, 2, 16))
    patches = out.reshape(B, N_PATCHES, PATCH, PATCH, C)
    return patches, samp


# speedup vs baseline: 1.4302x; 1.1353x over previous
import functools

import numpy as np
import jax
import jax.numpy as jnp
from jax import lax
from jax.experimental import pallas as pl
from jax.experimental.pallas import tpu as pltpu
from jax.experimental.pallas import tpu_sc as plsc

N_PATCHES = 64
PATCH = 32
B = 8
C = 32
HL = WL = 48
HH = WH = 384
EPS = 1e-8
NV = HL * WL

CELL = 16
CPR = WH // CELL
N_CELLS = B * C * HH * CPR
CELLS_PER_PATCH = C * PATCH * 3
PATCH_F32 = PATCH * PATCH * C

NC, NS = 2, 16
NW = NC * NS
PPW = (B * N_PATCHES) // NW

_k = np.arange(CELLS_PER_PATCH)
_PATTERN = ((_k // (PATCH * 3)) * (HH * CPR)
            + ((_k % (PATCH * 3)) // 3) * CPR + (_k % 3)).astype(np.int32)


def _sampler_body(att_ref, gum_ref, samp_ref, meta_ref):
    att = att_ref[...]
    scores = jnp.log(att + EPS) + gum_ref[...]
    pos = lax.broadcasted_iota(jnp.int32, (B, NV), 1)
    lane64 = lax.broadcasted_iota(jnp.int32, (B, N_PATCHES), 1)

    def step(i, carry):
        sc, fi, sa = carry
        m = jnp.max(sc, axis=1, keepdims=True)
        idx = jnp.min(jnp.where(sc == m, pos, NV), axis=1, keepdims=True)
        hit = pos == idx
        val = jnp.sum(jnp.where(hit, att, 0.0), axis=1, keepdims=True)
        onehot = lane64 == i
        fi = jnp.where(onehot, idx, fi)
        sa = jnp.where(onehot, val, sa)
        return jnp.where(hit, -jnp.inf, sc), fi, sa

    fi0 = jnp.zeros((B, N_PATCHES), jnp.int32)
    sa0 = jnp.zeros((B, N_PATCHES), jnp.float32)
    _, fi, sa = lax.fori_loop(0, N_PATCHES, step, (scores, fi0, sa0))
    samp_ref[...] = sa

    ys = fi // WL
    xs = fi - ys * WL
    top = jnp.clip(8 * ys - 12, 0, HH - PATCH)
    left = jnp.clip(8 * xs - 12, 0, WH - PATCH)
    bvec = lax.broadcasted_iota(jnp.int32, (B, N_PATCHES), 0)
    base = bvec * (C * HH * CPR) + top * CPR + left // CELL
    sh = left % CELL
    meta = jnp.stack(
        [jnp.broadcast_to(base[:, :, None], (B, N_PATCHES, 16)),
         jnp.broadcast_to(sh[:, :, None], (B, N_PATCHES, 16))], axis=2)
    meta_ref[...] = meta


def _run_sampler(att_flat, gumbel):
    return pl.pallas_call(
        _sampler_body,
        out_shape=(
            jax.ShapeDtypeStruct((B, N_PATCHES), jnp.float32),
            jax.ShapeDtypeStruct((B, N_PATCHES, 2, 16), jnp.int32),
        ),
    )(att_flat, gumbel)


def _gather_body(src_hbm, pat_hbm, meta_hbm, out_hbm,
                 pat_v, mv, idx_v, in_v, out_v, gsem, osem):
    wid = lax.axis_index("s") * NC + lax.axis_index("c")
    pltpu.sync_copy(pat_hbm, pat_v)
    lane = lax.iota(jnp.int32, 16)
    lane96 = lane * (PATCH * 3)
    clamp = jnp.full((16,), N_CELLS - 1, jnp.int32)

    def do_patch(t, _):
        p = wid * PPW + t
        slot = t % 2
        pltpu.sync_copy(meta_hbm.at[p], mv)
        base_splat = mv[0, :]
        sh_splat = mv[1, :]

        @plsc.parallel_loop(0, CELLS_PER_PATCH // 16, unroll=8)
        def build_idx(k):
            vals = jnp.minimum(pat_v[pl.ds(k * 16, 16)] + base_splat, clamp)
            idx_v[k // 8, pl.ds((k % 8) * 16, 16)] = vals

        cps = [
            pltpu.async_copy(src_hbm.at[idx_v.at[kk]],
                             in_v.at[pl.ds(kk * 128, 128)], gsem)
            for kk in range(CELLS_PER_PATCH // 128)
        ]
        for cp in cps:
            cp.wait()

        @pl.when(t >= 2)
        def _():
            pltpu.make_async_copy(
                out_hbm.at[p - 2], out_v.at[slot], osem.at[slot]).wait()

        @plsc.parallel_loop(0, PATCH * PATCH, unroll=8)
        def permute(ij):
            i = ij // PATCH
            j = ij - i * PATCH
            s2 = sh_splat + j
            q = s2 // CELL
            r = s2 - q * CELL
            row1 = lane96 + (i * 3) + q
            v1 = plsc.load_gather(in_v, [row1, r])
            v2 = plsc.load_gather(in_v, [row1 + 16 * PATCH * 3, r])
            off = i * (PATCH * C) + j * C
            out_v[slot, pl.ds(off, 16)] = v1
            out_v[slot, pl.ds(off + 16, 16)] = v2

        pltpu.async_copy(out_v.at[slot], out_hbm.at[p], osem.at[slot])
        return 0

    lax.fori_loop(0, PPW, do_patch, 0)
    pltpu.make_async_copy(out_hbm.at[0], out_v.at[0], osem.at[0]).wait()
    pltpu.make_async_copy(out_hbm.at[0], out_v.at[1], osem.at[1]).wait()


def _run_gather(src, pat, meta):
    mesh = plsc.VectorSubcoreMesh(core_axis_name="c", subcore_axis_name="s")
    k = functools.partial(
        pl.kernel,
        out_type=jax.ShapeDtypeStruct((B * N_PATCHES, PATCH_F32), jnp.float32),
        mesh=mesh,
        compiler_params=pltpu.CompilerParams(
            needs_layout_passes=False, use_tc_tiling_on_sc=False),
        scratch_types=[
            pltpu.VMEM((CELLS_PER_PATCH,), jnp.int32),
            pltpu.VMEM((2, 16), jnp.int32),
            pltpu.VMEM((CELLS_PER_PATCH // 128, 128), jnp.int32),
            pltpu.VMEM((CELLS_PER_PATCH, CELL), jnp.float32),
            pltpu.VMEM((2, PATCH_F32), jnp.float32),
            pltpu.SemaphoreType.DMA,
            pltpu.SemaphoreType.DMA((2,)),
        ],
    )(_gather_body)
    return k(src, pat, meta)


def kernel(x_low, x_high, attention, map_index):
    del x_low, map_index
    att_flat = attention.reshape(B, NV)
    u = jax.random.uniform(jax.random.key(42), (B, NV), minval=EPS, maxval=1.0)
    gumbel = -jnp.log(-jnp.log(u))

    samp, meta = _run_sampler(att_flat, gumbel)

    src = x_high.reshape(N_CELLS, CELL)
    out = _run_gather(src, jnp.asarray(_PATTERN),
                      meta.reshape(B * N_PATCHES, 2, 16))
    patches = out.reshape(B, N_PATCHES, PATCH, PATCH, C)
    return patches, samp


# speedup vs baseline: 1.4395x; 1.0065x over previous
import functools

import numpy as np
import jax
import jax.numpy as jnp
from jax import lax
from jax.experimental import pallas as pl
from jax.experimental.pallas import tpu as pltpu
from jax.experimental.pallas import tpu_sc as plsc

N_PATCHES = 64
PATCH = 32
B = 8
C = 32
HL = WL = 48
HH = WH = 384
EPS = 1e-8
NV = HL * WL

CELL = 16
CPR = WH // CELL
N_CELLS = B * C * HH * CPR
CELLS_PER_PATCH = C * PATCH * 3
PATCH_F32 = PATCH * PATCH * C

NC, NS = 2, 16
NW = NC * NS
PPW = (B * N_PATCHES) // NW

_k = np.arange(CELLS_PER_PATCH)
_PATTERN = ((_k // (PATCH * 3)) * (HH * CPR)
            + ((_k % (PATCH * 3)) // 3) * CPR + (_k % 3)).astype(np.int32)


def _sampler_body(att_ref, gum_ref, samp_ref, meta_ref):
    att = att_ref[...]
    scores = jnp.log(att + EPS) + gum_ref[...]
    pos = lax.broadcasted_iota(jnp.int32, (B, NV), 1)
    lane64 = lax.broadcasted_iota(jnp.int32, (B, N_PATCHES), 1)

    def step(i, carry):
        sc, fi, sa = carry
        m = jnp.max(sc, axis=1, keepdims=True)
        idx = jnp.min(jnp.where(sc == m, pos, NV), axis=1, keepdims=True)
        hit = pos == idx
        val = jnp.sum(jnp.where(hit, att, 0.0), axis=1, keepdims=True)
        onehot = lane64 == i
        fi = jnp.where(onehot, idx, fi)
        sa = jnp.where(onehot, val, sa)
        return jnp.where(hit, -jnp.inf, sc), fi, sa

    fi0 = jnp.zeros((B, N_PATCHES), jnp.int32)
    sa0 = jnp.zeros((B, N_PATCHES), jnp.float32)
    _, fi, sa = lax.fori_loop(0, N_PATCHES, step, (scores, fi0, sa0))
    samp_ref[...] = sa

    ys = fi // WL
    xs = fi - ys * WL
    top = jnp.clip(8 * ys - 12, 0, HH - PATCH)
    left = jnp.clip(8 * xs - 12, 0, WH - PATCH)
    bvec = lax.broadcasted_iota(jnp.int32, (B, N_PATCHES), 0)
    base = bvec * (C * HH * CPR) + top * CPR + left // CELL
    sh = left % CELL
    meta = jnp.stack(
        [jnp.broadcast_to(base[:, :, None], (B, N_PATCHES, 16)),
         jnp.broadcast_to(sh[:, :, None], (B, N_PATCHES, 16))], axis=2)
    meta_ref[...] = meta


def _run_sampler(att_flat, gumbel):
    return pl.pallas_call(
        _sampler_body,
        out_shape=(
            jax.ShapeDtypeStruct((B, N_PATCHES), jnp.float32),
            jax.ShapeDtypeStruct((B, N_PATCHES, 2, 16), jnp.int32),
        ),
    )(att_flat, gumbel)


HCELLS = CELLS_PER_PATCH // 2


def _gather_body(src_hbm, pat_hbm, meta_hbm, out_hbm,
                 pat_v, mvs, idx_v, in_v, out_v, gsem, osem):
    wid = lax.axis_index("s") * NC + lax.axis_index("c")
    p0 = wid * PPW
    pltpu.sync_copy(pat_hbm, pat_v)
    lane = lax.iota(jnp.int32, 16)
    lane96 = lane * (PATCH * 3)
    clamp = jnp.full((16,), N_CELLS - 1, jnp.int32)

    def build_fire(un):
        hn = un & 1
        tn = un >> 1
        base = mvs[tn & 1, 0, :]

        @plsc.parallel_loop(0, HCELLS // 16, unroll=8)
        def bi(k):
            vals = jnp.minimum(
                pat_v[pl.ds(hn * HCELLS + k * 16, 16)] + base, clamp)
            idx_v[hn, k // 8, pl.ds((k % 8) * 16, 16)] = vals

        for kk in range(HCELLS // 128):
            pltpu.async_copy(src_hbm.at[idx_v.at[hn, kk]],
                             in_v.at[pl.ds(hn * HCELLS + kk * 128, 128)],
                             gsem.at[hn])

    pltpu.sync_copy(meta_hbm.at[p0], mvs.at[0])
    build_fire(0)

    def unit_iter(u, _):
        h = u & 1
        t = u >> 1
        ts = t & 1
        sh = mvs[ts, 1, :]

        @pl.when(u < 2 * PPW - 1)
        def _():
            @pl.when(h == 1)
            def _():
                pltpu.sync_copy(meta_hbm.at[p0 + t + 1], mvs.at[(t + 1) & 1])
            build_fire(u + 1)

        pltpu.make_async_copy(src_hbm.at[pl.ds(0, HCELLS)],
                              in_v.at[pl.ds(h * HCELLS, HCELLS)],
                              gsem.at[h]).wait()

        @pl.when((h == 0) & (t >= 2))
        def _():
            pltpu.make_async_copy(out_hbm.at[0], out_v.at[ts],
                                  osem.at[ts]).wait()

        @plsc.parallel_loop(0, PATCH * PATCH, unroll=8)
        def permute(ij):
            i = ij // PATCH
            j = ij - i * PATCH
            s2 = sh + j
            q = s2 // CELL
            r = s2 - q * CELL
            row = lane96 + (h * HCELLS + i * 3) + q
            v = plsc.load_gather(in_v, [row, r])
            out_v[ts, pl.ds(i * (PATCH * C) + j * C + h * 16, 16)] = v

        @pl.when(h == 1)
        def _():
            pltpu.async_copy(out_v.at[ts], out_hbm.at[p0 + t], osem.at[ts])
        return 0

    lax.fori_loop(0, 2 * PPW, unit_iter, 0)
    pltpu.make_async_copy(out_hbm.at[0], out_v.at[0], osem.at[0]).wait()
    pltpu.make_async_copy(out_hbm.at[0], out_v.at[1], osem.at[1]).wait()


def _run_gather(src, pat, meta):
    mesh = plsc.VectorSubcoreMesh(core_axis_name="c", subcore_axis_name="s")
    k = functools.partial(
        pl.kernel,
        out_type=jax.ShapeDtypeStruct((B * N_PATCHES, PATCH_F32), jnp.float32),
        mesh=mesh,
        compiler_params=pltpu.CompilerParams(
            needs_layout_passes=False, use_tc_tiling_on_sc=False),
        scratch_types=[
            pltpu.VMEM((CELLS_PER_PATCH,), jnp.int32),
            pltpu.VMEM((2, 2, 16), jnp.int32),
            pltpu.VMEM((2, HCELLS // 128, 128), jnp.int32),
            pltpu.VMEM((CELLS_PER_PATCH, CELL), jnp.float32),
            pltpu.VMEM((2, PATCH_F32), jnp.float32),
            pltpu.SemaphoreType.DMA((2,)),
            pltpu.SemaphoreType.DMA((2,)),
        ],
    )(_gather_body)
    return k(src, pat, meta)


def kernel(x_low, x_high, attention, map_index):
    del x_low, map_index
    att_flat = attention.reshape(B, NV)
    u = jax.random.uniform(jax.random.key(42), (B, NV), minval=EPS, maxval=1.0)
    gumbel = -jnp.log(-jnp.log(u))

    samp, meta = _run_sampler(att_flat, gumbel)

    src = x_high.reshape(N_CELLS, CELL)
    out = _run_gather(src, jnp.asarray(_PATTERN),
                      meta.reshape(B * N_PATCHES, 2, 16))
    patches = out.reshape(B, N_PATCHES, PATCH, PATCH, C)
    return patches, samp


# speedup vs baseline: 1.6397x; 1.1391x over previous
import functools

import numpy as np
import jax
import jax.numpy as jnp
from jax import lax
from jax.experimental import pallas as pl
from jax.experimental.pallas import tpu as pltpu
from jax.experimental.pallas import tpu_sc as plsc

N_PATCHES = 64
PATCH = 32
B = 8
C = 32
HL = WL = 48
HH = WH = 384
EPS = 1e-8
NV = HL * WL

CELL = 16
CPR = WH // CELL
N_CELLS = B * C * HH * CPR
CELLS_PER_PATCH = C * PATCH * 3
PATCH_F32 = PATCH * PATCH * C

NC, NS = 2, 16
NW = NC * NS
PPW = (B * N_PATCHES) // NW

_k = np.arange(CELLS_PER_PATCH)
_PATTERN = ((_k // (PATCH * 3)) * (HH * CPR)
            + ((_k % (PATCH * 3)) // 3) * CPR + (_k % 3)).astype(np.int32)


def _sampler_body(att_ref, gum_ref, samp_ref, meta_ref):
    att = att_ref[...]
    scores = jnp.log(att + EPS) + gum_ref[...]
    pos = lax.broadcasted_iota(jnp.int32, (B, NV), 1)
    lane64 = lax.broadcasted_iota(jnp.int32, (B, N_PATCHES), 1)

    def step(i, carry):
        sc, fi, sa = carry
        m = jnp.max(sc, axis=1, keepdims=True)
        idx = jnp.min(jnp.where(sc == m, pos, NV), axis=1, keepdims=True)
        hit = pos == idx
        val = jnp.sum(jnp.where(hit, att, 0.0), axis=1, keepdims=True)
        onehot = lane64 == i
        fi = jnp.where(onehot, idx, fi)
        sa = jnp.where(onehot, val, sa)
        return jnp.where(hit, -jnp.inf, sc), fi, sa

    fi0 = jnp.zeros((B, N_PATCHES), jnp.int32)
    sa0 = jnp.zeros((B, N_PATCHES), jnp.float32)
    _, fi, sa = lax.fori_loop(0, N_PATCHES, step, (scores, fi0, sa0))
    samp_ref[...] = sa

    ys = fi // WL
    xs = fi - ys * WL
    top = jnp.clip(8 * ys - 12, 0, HH - PATCH)
    left = jnp.clip(8 * xs - 12, 0, WH - PATCH)
    bvec = lax.broadcasted_iota(jnp.int32, (B, N_PATCHES), 0)
    base = bvec * (C * HH * CPR) + top * CPR + left // CELL
    sh = left % CELL
    meta = jnp.stack(
        [jnp.broadcast_to(base[:, :, None], (B, N_PATCHES, 16)),
         jnp.broadcast_to(sh[:, :, None], (B, N_PATCHES, 16))], axis=2)
    meta_ref[...] = meta


def _run_sampler(att_flat, gumbel):
    return pl.pallas_call(
        _sampler_body,
        out_shape=(
            jax.ShapeDtypeStruct((B, N_PATCHES), jnp.float32),
            jax.ShapeDtypeStruct((B, N_PATCHES, 2, 16), jnp.int32),
        ),
    )(att_flat, gumbel)


HCELLS = CELLS_PER_PATCH // 2


def _gather_body(src_hbm, pat_hbm, meta_hbm, out_hbm,
                 pat_v, mvs, idx_v, in_v, out_v, gsem, osem):
    wid = lax.axis_index("s") * NC + lax.axis_index("c")
    p0 = wid * PPW
    pltpu.sync_copy(pat_hbm, pat_v)
    lane = lax.iota(jnp.int32, 16)
    lane96 = lane * (PATCH * 3)
    clamp = jnp.full((16,), N_CELLS - 1, jnp.int32)

    def build_fire(un):
        hn = un & 1
        tn = un >> 1
        base = mvs[tn & 1, 0, :]

        @plsc.parallel_loop(0, HCELLS // 16, unroll=8)
        def bi(k):
            vals = jnp.minimum(
                pat_v[pl.ds(hn * HCELLS + k * 16, 16)] + base, clamp)
            idx_v[hn, k // 8, pl.ds((k % 8) * 16, 16)] = vals

        for kk in range(HCELLS // 128):
            pltpu.async_copy(src_hbm.at[idx_v.at[hn, kk]],
                             in_v.at[pl.ds(hn * HCELLS + kk * 128, 128)],
                             gsem.at[hn])

    pltpu.sync_copy(meta_hbm.at[p0], mvs.at[0])
    build_fire(0)

    def unit_iter(u, _):
        h = u & 1
        t = u >> 1
        ts = t & 1
        sh = mvs[ts, 1, :]

        @pl.when(u < 2 * PPW - 1)
        def _():
            @pl.when(h == 1)
            def _():
                pltpu.sync_copy(meta_hbm.at[p0 + t + 1], mvs.at[(t + 1) & 1])
            build_fire(u + 1)

        pltpu.make_async_copy(src_hbm.at[pl.ds(0, HCELLS)],
                              in_v.at[pl.ds(h * HCELLS, HCELLS)],
                              gsem.at[h]).wait()

        @pl.when((h == 0) & (t >= 2))
        def _():
            pltpu.make_async_copy(out_hbm.at[0], out_v.at[ts],
                                  osem.at[ts]).wait()

        lane_sh = lane96 + h * HCELLS

        @plsc.parallel_loop(0, PATCH * PATCH, unroll=8)
        def permute(ij):
            i = ij // PATCH
            j = ij - i * PATCH
            row = lane_sh + i * 3
            v = plsc.load_gather(in_v, [row, sh + j])
            out_v[ts, pl.ds(i * (PATCH * C) + j * C + h * 16, 16)] = v

        @pl.when(h == 1)
        def _():
            pltpu.async_copy(out_v.at[ts], out_hbm.at[p0 + t], osem.at[ts])
        return 0

    lax.fori_loop(0, 2 * PPW, unit_iter, 0)
    pltpu.make_async_copy(out_hbm.at[0], out_v.at[0], osem.at[0]).wait()
    pltpu.make_async_copy(out_hbm.at[0], out_v.at[1], osem.at[1]).wait()


def _run_gather(src, pat, meta):
    mesh = plsc.VectorSubcoreMesh(core_axis_name="c", subcore_axis_name="s")
    k = functools.partial(
        pl.kernel,
        out_type=jax.ShapeDtypeStruct((B * N_PATCHES, PATCH_F32), jnp.float32),
        mesh=mesh,
        compiler_params=pltpu.CompilerParams(
            needs_layout_passes=False, use_tc_tiling_on_sc=False),
        scratch_types=[
            pltpu.VMEM((CELLS_PER_PATCH,), jnp.int32),
            pltpu.VMEM((2, 2, 16), jnp.int32),
            pltpu.VMEM((2, HCELLS // 128, 128), jnp.int32),
            pltpu.VMEM((CELLS_PER_PATCH, CELL), jnp.float32),
            pltpu.VMEM((2, PATCH_F32), jnp.float32),
            pltpu.SemaphoreType.DMA((2,)),
            pltpu.SemaphoreType.DMA((2,)),
        ],
    )(_gather_body)
    return k(src, pat, meta)


def kernel(x_low, x_high, attention, map_index):
    del x_low, map_index
    att_flat = attention.reshape(B, NV)
    u = jax.random.uniform(jax.random.key(42), (B, NV), minval=EPS, maxval=1.0)
    gumbel = -jnp.log(-jnp.log(u))

    samp, meta = _run_sampler(att_flat, gumbel)

    src = x_high.reshape(N_CELLS, CELL)
    out = _run_gather(src, jnp.asarray(_PATTERN),
                      meta.reshape(B * N_PATCHES, 2, 16))
    patches = out.reshape(B, N_PATCHES, PATCH, PATCH, C)
    return patches, samp


# speedup vs baseline: 1.6575x; 1.0108x over previous
import functools

import numpy as np
import jax
import jax.numpy as jnp
from jax import lax
from jax.experimental import pallas as pl
from jax.experimental.pallas import tpu as pltpu
from jax.experimental.pallas import tpu_sc as plsc

N_PATCHES = 64
PATCH = 32
B = 8
C = 32
HL = WL = 48
HH = WH = 384
EPS = 1e-8
NV = HL * WL

CELL = 16
CPR = WH // CELL
N_CELLS = B * C * HH * CPR
CELLS_PER_PATCH = C * PATCH * 3
PATCH_F32 = PATCH * PATCH * C

NC, NS = 2, 16
NW = NC * NS
PPW = (B * N_PATCHES) // NW

_k = np.arange(CELLS_PER_PATCH)
_PATTERN = ((_k // (PATCH * 3)) * (HH * CPR)
            + ((_k % (PATCH * 3)) // 3) * CPR + (_k % 3)).astype(np.int32)


def _sampler_body(att_ref, gum_ref, samp_ref, meta_ref):
    att = att_ref[...]
    scores = jnp.log(att + EPS) + gum_ref[...]
    pos = lax.broadcasted_iota(jnp.int32, (B, NV), 1)
    lane64 = lax.broadcasted_iota(jnp.int32, (B, N_PATCHES), 1)

    def step(i, carry):
        sc, fi, sa = carry
        m = jnp.max(sc, axis=1, keepdims=True)
        idx = jnp.min(jnp.where(sc == m, pos, NV), axis=1, keepdims=True)
        hit = pos == idx
        val = jnp.sum(jnp.where(hit, att, 0.0), axis=1, keepdims=True)
        onehot = lane64 == i
        fi = jnp.where(onehot, idx, fi)
        sa = jnp.where(onehot, val, sa)
        return jnp.where(hit, -jnp.inf, sc), fi, sa

    fi0 = jnp.zeros((B, N_PATCHES), jnp.int32)
    sa0 = jnp.zeros((B, N_PATCHES), jnp.float32)
    _, fi, sa = lax.fori_loop(0, N_PATCHES, step, (scores, fi0, sa0))
    samp_ref[...] = sa

    ys = fi // WL
    xs = fi - ys * WL
    top = jnp.clip(8 * ys - 12, 0, HH - PATCH)
    left = jnp.clip(8 * xs - 12, 0, WH - PATCH)
    bvec = lax.broadcasted_iota(jnp.int32, (B, N_PATCHES), 0)
    base = bvec * (C * HH * CPR) + top * CPR + left // CELL
    sh = left % CELL
    meta = jnp.stack(
        [jnp.broadcast_to(base[:, :, None], (B, N_PATCHES, 16)),
         jnp.broadcast_to(sh[:, :, None], (B, N_PATCHES, 16))], axis=2)
    meta_ref[...] = meta


def _run_sampler(att_flat, gumbel):
    return pl.pallas_call(
        _sampler_body,
        out_shape=(
            jax.ShapeDtypeStruct((B, N_PATCHES), jnp.float32),
            jax.ShapeDtypeStruct((B, N_PATCHES, 2, 16), jnp.int32),
        ),
    )(att_flat, gumbel)


HCELLS = CELLS_PER_PATCH // 2


def _gather_body(src_hbm, pat_hbm, meta_hbm, out_hbm,
                 pat_v, mvs, idx_v, in_v, out_v, gsem, osem):
    wid = lax.axis_index("s") * NC + lax.axis_index("c")
    p0 = wid * PPW
    pltpu.sync_copy(pat_hbm, pat_v)
    lane = lax.iota(jnp.int32, 16)
    lane96 = lane * (PATCH * 3)
    clamp = jnp.full((16,), N_CELLS - 1, jnp.int32)

    def build_fire(un):
        hn = un & 1
        tn = un >> 1
        base = mvs[tn & 1, 0, :]

        @plsc.parallel_loop(0, HCELLS // 16, unroll=8)
        def bi(k):
            vals = jnp.minimum(
                pat_v[pl.ds(hn * HCELLS + k * 16, 16)] + base, clamp)
            idx_v[hn, k // 8, pl.ds((k % 8) * 16, 16)] = vals

        for kk in range(HCELLS // 128):
            pltpu.async_copy(src_hbm.at[idx_v.at[hn, kk]],
                             in_v.at[pl.ds(hn * HCELLS + kk * 128, 128)],
                             gsem.at[hn])

    pltpu.sync_copy(meta_hbm.at[p0], mvs.at[0])
    build_fire(0)

    def unit_iter(u, _):
        h = u & 1
        t = u >> 1
        ts = t & 1
        sh = mvs[ts, 1, :]

        @pl.when(u < 2 * PPW - 1)
        def _():
            @pl.when(h == 1)
            def _():
                pltpu.sync_copy(meta_hbm.at[p0 + t + 1], mvs.at[(t + 1) & 1])
            build_fire(u + 1)

        pltpu.make_async_copy(src_hbm.at[pl.ds(0, HCELLS)],
                              in_v.at[pl.ds(h * HCELLS, HCELLS)],
                              gsem.at[h]).wait()

        @pl.when((h == 0) & (t >= 2))
        def _():
            pltpu.make_async_copy(out_hbm.at[0], out_v.at[ts],
                                  osem.at[ts]).wait()

        lane_sh = lane96 + h * HCELLS

        @plsc.parallel_loop(0, PATCH * PATCH, unroll=16)
        def permute(ij):
            i = ij // PATCH
            j = ij - i * PATCH
            row = lane_sh + i * 3
            v = plsc.load_gather(in_v, [row, sh + j])
            out_v[ts, pl.ds(i * (PATCH * C) + j * C + h * 16, 16)] = v

        @pl.when(h == 1)
        def _():
            pltpu.async_copy(out_v.at[ts], out_hbm.at[p0 + t], osem.at[ts])
        return 0

    lax.fori_loop(0, 2 * PPW, unit_iter, 0)
    pltpu.make_async_copy(out_hbm.at[0], out_v.at[0], osem.at[0]).wait()
    pltpu.make_async_copy(out_hbm.at[0], out_v.at[1], osem.at[1]).wait()


def _run_gather(src, pat, meta):
    mesh = plsc.VectorSubcoreMesh(core_axis_name="c", subcore_axis_name="s")
    k = functools.partial(
        pl.kernel,
        out_type=jax.ShapeDtypeStruct((B * N_PATCHES, PATCH_F32), jnp.float32),
        mesh=mesh,
        compiler_params=pltpu.CompilerParams(
            needs_layout_passes=False, use_tc_tiling_on_sc=False),
        scratch_types=[
            pltpu.VMEM((CELLS_PER_PATCH,), jnp.int32),
            pltpu.VMEM((2, 2, 16), jnp.int32),
            pltpu.VMEM((2, HCELLS // 128, 128), jnp.int32),
            pltpu.VMEM((CELLS_PER_PATCH, CELL), jnp.float32),
            pltpu.VMEM((2, PATCH_F32), jnp.float32),
            pltpu.SemaphoreType.DMA((2,)),
            pltpu.SemaphoreType.DMA((2,)),
        ],
    )(_gather_body)
    return k(src, pat, meta)


def kernel(x_low, x_high, attention, map_index):
    del x_low, map_index
    att_flat = attention.reshape(B, NV)
    u = jax.random.uniform(jax.random.key(42), (B, NV), minval=EPS, maxval=1.0)
    gumbel = -jnp.log(-jnp.log(u))

    samp, meta = _run_sampler(att_flat, gumbel)

    src = x_high.reshape(N_CELLS, CELL)
    out = _run_gather(src, jnp.asarray(_PATTERN),
                      meta.reshape(B * N_PATCHES, 2, 16))
    patches = out.reshape(B, N_PATCHES, PATCH, PATCH, C)
    return patches, samp


# speedup vs baseline: 2.0084x; 1.2118x over previous
import functools

import numpy as np
import jax
import jax.numpy as jnp
from jax import lax
from jax.experimental import pallas as pl
from jax.experimental.pallas import tpu as pltpu
from jax.experimental.pallas import tpu_sc as plsc

N_PATCHES = 64
PATCH = 32
B = 8
C = 32
HL = WL = 48
HH = WH = 384
EPS = 1e-8
NV = HL * WL

CELL = 16
CPR = WH // CELL
N_CELLS = B * C * HH * CPR
CELLS_PER_PATCH = C * PATCH * 3
PATCH_F32 = PATCH * PATCH * C

NC, NS = 2, 16
NW = NC * NS
PPW = (B * N_PATCHES) // NW

_k = np.arange(CELLS_PER_PATCH)
_PATTERN = ((_k // (PATCH * 3)) * (HH * CPR)
            + ((_k % (PATCH * 3)) // 3) * CPR + (_k % 3)).astype(np.int32)


def _sampler_body(att_ref, gum_ref, samp_ref, meta_ref, tau_ref):
    att = att_ref[...]
    scores = jnp.log(att + EPS) + gum_ref[...]
    pos = lax.broadcasted_iota(jnp.int32, (B, NV), 1)
    lane64 = lax.broadcasted_iota(jnp.int32, (B, N_PATCHES), 1)

    def step(i, carry):
        sc, fi, sa = carry
        m = jnp.max(sc, axis=1, keepdims=True)
        idx = jnp.min(jnp.where(sc == m, pos, NV), axis=1, keepdims=True)
        hit = pos == idx
        val = jnp.sum(jnp.where(hit, att, 0.0), axis=1, keepdims=True)
        onehot = lane64 == i
        fi = jnp.where(onehot, idx, fi)
        sa = jnp.where(onehot, val, sa)
        return jnp.where(hit, -jnp.inf, sc), fi, sa

    fi0 = jnp.zeros((B, N_PATCHES), jnp.int32)
    sa0 = jnp.zeros((B, N_PATCHES), jnp.float32)
    _, fi, sa = lax.fori_loop(0, N_PATCHES, step, (scores, fi0, sa0))
    samp_ref[...] = sa

    ys = fi // WL
    xs = fi - ys * WL
    top = jnp.clip(8 * ys - 12, 0, HH - PATCH)
    left = jnp.clip(8 * xs - 12, 0, WH - PATCH)
    bvec = lax.broadcasted_iota(jnp.int32, (B, N_PATCHES), 0)
    base = bvec * (C * HH * CPR)
    sh = left % CELL
    meta = jnp.stack(
        [jnp.broadcast_to(base[:, :, None], (B, N_PATCHES, 16)),
         jnp.broadcast_to(sh[:, :, None], (B, N_PATCHES, 16))], axis=2)
    meta_ref[...] = meta
    k = lax.broadcasted_iota(jnp.int32, (B, N_PATCHES, PATCH * 3), 2)
    hh = top[:, :, None] + k // 3
    wc16 = jnp.minimum(left[:, :, None] // CELL + k % 3, WH // CELL - 1)
    tau_ref[...] = (((hh // 8) * 3 + wc16 // 8) * 8 + hh % 8) * 8 + wc16 % 8


def _run_sampler(att_flat, gumbel):
    return pl.pallas_call(
        _sampler_body,
        out_shape=(
            jax.ShapeDtypeStruct((B, N_PATCHES), jnp.float32),
            jax.ShapeDtypeStruct((B, N_PATCHES, 2, 16), jnp.int32),
            jax.ShapeDtypeStruct((B, N_PATCHES, PATCH * 3), jnp.int32),
        ),
    )(att_flat, gumbel)


HCELLS = CELLS_PER_PATCH // 2


def _gather_body(src_hbm, meta_hbm, tau_hbm, out_hbm,
                 mvs, tau_v, idx_v, in_v, out_v, gsem, osem):
    wid = lax.axis_index("s") * NC + lax.axis_index("c")
    p0 = wid * PPW
    lane = lax.iota(jnp.int32, 16)
    lane96 = lane * (PATCH * 3)

    def build_fire(un):
        hn = un & 1
        tn = un >> 1
        ts2 = tn & 1
        bbase = mvs[ts2, 0, :]

        @plsc.parallel_loop(0, HCELLS // 16, unroll=8)
        def bi(k):
            cl = k // 6
            tc = k - cl * 6
            vals = (tau_v[ts2, pl.ds(tc * 16, 16)] + bbase
                    + (hn * 16 + cl) * (HH * CPR))
            m = cl * 96 + tc * 16
            idx_v[hn, m // 128, pl.ds(m % 128, 16)] = vals

        for kk in range(HCELLS // 128):
            pltpu.async_copy(src_hbm.at[idx_v.at[hn, kk]],
                             in_v.at[pl.ds(hn * HCELLS + kk * 128, 128)],
                             gsem.at[hn])

    pltpu.sync_copy(meta_hbm.at[p0], mvs.at[0])
    pltpu.sync_copy(tau_hbm.at[p0], tau_v.at[0])
    build_fire(0)

    def unit_iter(u, _):
        h = u & 1
        t = u >> 1
        ts = t & 1
        sh = mvs[ts, 1, :]

        @pl.when(u < 2 * PPW - 1)
        def _():
            @pl.when(h == 1)
            def _():
                pltpu.sync_copy(meta_hbm.at[p0 + t + 1], mvs.at[(t + 1) & 1])
                pltpu.sync_copy(tau_hbm.at[p0 + t + 1], tau_v.at[(t + 1) & 1])
            build_fire(u + 1)

        pltpu.make_async_copy(src_hbm.at[pl.ds(0, HCELLS)],
                              in_v.at[pl.ds(h * HCELLS, HCELLS)],
                              gsem.at[h]).wait()

        @pl.when((h == 0) & (t >= 2))
        def _():
            pltpu.make_async_copy(out_hbm.at[0], out_v.at[ts],
                                  osem.at[ts]).wait()

        lane_sh = lane96 + h * HCELLS

        @plsc.parallel_loop(0, PATCH * PATCH, unroll=16)
        def permute(ij):
            i = ij // PATCH
            j = ij - i * PATCH
            row = lane_sh + i * 3
            v = plsc.load_gather(in_v, [row, sh + j])
            out_v[ts, pl.ds(i * (PATCH * C) + j * C + h * 16, 16)] = v

        @pl.when(h == 1)
        def _():
            pltpu.async_copy(out_v.at[ts], out_hbm.at[p0 + t], osem.at[ts])
        return 0

    lax.fori_loop(0, 2 * PPW, unit_iter, 0)
    pltpu.make_async_copy(out_hbm.at[0], out_v.at[0], osem.at[0]).wait()
    pltpu.make_async_copy(out_hbm.at[0], out_v.at[1], osem.at[1]).wait()


def _run_gather(src, meta, tau):
    mesh = plsc.VectorSubcoreMesh(core_axis_name="c", subcore_axis_name="s")
    k = functools.partial(
        pl.kernel,
        out_type=jax.ShapeDtypeStruct((B * N_PATCHES, PATCH_F32), jnp.float32),
        mesh=mesh,
        compiler_params=pltpu.CompilerParams(
            needs_layout_passes=False, use_tc_tiling_on_sc=False),
        scratch_types=[
            pltpu.VMEM((2, 2, 16), jnp.int32),
            pltpu.VMEM((2, PATCH * 3), jnp.int32),
            pltpu.VMEM((2, HCELLS // 128, 128), jnp.int32),
            pltpu.VMEM((CELLS_PER_PATCH, CELL), jnp.float32),
            pltpu.VMEM((2, PATCH_F32), jnp.float32),
            pltpu.SemaphoreType.DMA((2,)),
            pltpu.SemaphoreType.DMA((2,)),
        ],
    )(_gather_body)
    return k(src, meta, tau)


def kernel(x_low, x_high, attention, map_index):
    del x_low, map_index
    att_flat = attention.reshape(B, NV)
    u = jax.random.uniform(jax.random.key(42), (B, NV), minval=EPS, maxval=1.0)
    gumbel = -jnp.log(-jnp.log(u))

    samp, meta, tau = _run_sampler(att_flat, gumbel)

    xt = jnp.transpose(x_high.reshape(B, C, HH // 8, 8, WH // 128, 128),
                       (0, 1, 2, 4, 3, 5))
    src = xt.reshape(N_CELLS, CELL)
    out = _run_gather(src, meta.reshape(B * N_PATCHES, 2, 16),
                      tau.reshape(B * N_PATCHES, PATCH * 3))
    patches = out.reshape(B, N_PATCHES, PATCH, PATCH, C)
    return patches, samp


# speedup vs baseline: 3.7595x; 1.8718x over previous
import functools

import numpy as np
import jax
import jax.numpy as jnp
from jax import lax
from jax.experimental import pallas as pl
from jax.experimental.pallas import tpu as pltpu
from jax.experimental.pallas import tpu_sc as plsc

N_PATCHES = 64
PATCH = 32
B = 8
C = 32
HL = WL = 48
HH = WH = 384
EPS = 1e-8
NV = HL * WL

CELL = 16
CPR = WH // CELL
PLANE = HH * CPR
N_CELLS = B * C * PLANE
PATCH_F32 = PATCH * PATCH * C

NC, NS = 2, 16
NW = NC * NS
CQ = 4
UCELLS = N_PATCHES * (C // CQ) * 3

_mm = np.arange(192)
_TPAT = np.stack([_mm % 3, _mm // 3]).astype(np.int32)


def _sampler_body(att_ref, gum_ref, samp_ref, taut_ref, sh_ref):
    att = att_ref[...]
    scores = jnp.log(att + EPS) + gum_ref[...]
    pos = lax.broadcasted_iota(jnp.int32, (B, NV), 1)
    lane64 = lax.broadcasted_iota(jnp.int32, (B, N_PATCHES), 1)

    def step(i, carry):
        sc, fi, sa = carry
        m = jnp.max(sc, axis=1, keepdims=True)
        idx = jnp.min(jnp.where(sc == m, pos, NV), axis=1, keepdims=True)
        hit = pos == idx
        val = jnp.sum(jnp.where(hit, att, 0.0), axis=1, keepdims=True)
        onehot = lane64 == i
        fi = jnp.where(onehot, idx, fi)
        sa = jnp.where(onehot, val, sa)
        return jnp.where(hit, -jnp.inf, sc), fi, sa

    fi0 = jnp.zeros((B, N_PATCHES), jnp.int32)
    sa0 = jnp.zeros((B, N_PATCHES), jnp.float32)
    _, fi, sa = lax.fori_loop(0, N_PATCHES, step, (scores, fi0, sa0))
    samp_ref[...] = sa

    ys = fi // WL
    xs = fi - ys * WL
    top = jnp.clip(8 * ys - 12, 0, HH - PATCH)
    left = jnp.clip(8 * xs - 12, 0, WH - PATCH)
    sh_ref[...] = left % CELL
    k = lax.broadcasted_iota(jnp.int32, (B, PATCH * 3, N_PATCHES), 1)
    topB = jnp.broadcast_to(top[:, None, :], (B, PATCH * 3, N_PATCHES))
    leftB = jnp.broadcast_to(left[:, None, :], (B, PATCH * 3, N_PATCHES))
    hh = topB + k // 3
    wc16 = jnp.minimum(leftB // CELL + k % 3, CPR - 1)
    taut_ref[...] = (((hh // 8) * 3 + wc16 // 8) * 8 + hh % 8) * 8 + wc16 % 8


def _run_sampler(att_flat, gumbel):
    return pl.pallas_call(
        _sampler_body,
        out_shape=(
            jax.ShapeDtypeStruct((B, N_PATCHES), jnp.float32),
            jax.ShapeDtypeStruct((B, PATCH * 3, N_PATCHES), jnp.int32),
            jax.ShapeDtypeStruct((B, N_PATCHES), jnp.int32),
        ),
    )(att_flat, gumbel)


def _gather_body(src_hbm, taut_hbm, sh_hbm, tpat_hbm, out_hbm,
                 tpat_v, sh_v, tau3_v, idx_v, in_v, out_v, gsem, osem):
    wid = lax.axis_index("s") * NC + lax.axis_index("c")
    b = wid // 4
    ig = wid - b * 4
    bbase = b * (C * PLANE)
    pltpu.sync_copy(tpat_hbm, tpat_v)
    pltpu.sync_copy(sh_hbm.at[b], sh_v)
    lane = lax.iota(jnp.int32, 16)

    def do_row(iu, _):
        i = ig * 8 + iu
        pltpu.sync_copy(taut_hbm.at[b, pl.ds(i * 3, 3)], tau3_v)

        @pl.when(iu >= 1)
        def _():
            pltpu.make_async_copy(out_hbm.at[b, pl.ds(0, PATCH * C)],
                                  out_v, osem).wait()

        for cq in range(CQ):

            @plsc.parallel_loop(0, UCELLS // 16, unroll=8)
            def build(kk):
                cl = kk // 12
                mmc = (kk - cl * 12) * 16
                tvals = plsc.load_gather(
                    tau3_v, [tpat_v[0, pl.ds(mmc, 16)],
                             tpat_v[1, pl.ds(mmc, 16)]])
                vals = tvals + (bbase + (cq * 8 + cl) * PLANE)
                idx_v[kk // 8, pl.ds((kk % 8) * 16, 16)] = vals

            for kk in range(UCELLS // 128):
                pltpu.async_copy(src_hbm.at[idx_v.at[kk]],
                                 in_v.at[pl.ds(kk * 128, 128)], gsem)
            pltpu.make_async_copy(src_hbm.at[pl.ds(0, UCELLS)], in_v,
                                  gsem).wait()

            for g in range(4):
                rowg = lane * 3 + g * 48
                colg = sh_v[pl.ds(g * 16, 16)]

                @plsc.parallel_loop(0, 256, unroll=8)
                def permute(cj):
                    cl = cj // PATCH
                    j = cj - cl * PATCH
                    v = plsc.load_gather(in_v, [rowg + cl * 192, colg + j])
                    out_v[j * C + cq * 8 + cl, pl.ds(g * 16, 16)] = v

        pltpu.async_copy(out_v, out_hbm.at[b, pl.ds(i * (PATCH * C),
                                                    PATCH * C)], osem)
        return 0

    lax.fori_loop(0, 8, do_row, 0)
    pltpu.make_async_copy(out_hbm.at[0, pl.ds(0, PATCH * C)],
                          out_v, osem).wait()


def _run_gather(src, taut, sh, tpat):
    mesh = plsc.VectorSubcoreMesh(core_axis_name="c", subcore_axis_name="s")
    k = functools.partial(
        pl.kernel,
        out_type=jax.ShapeDtypeStruct((B, PATCH * PATCH * C, N_PATCHES),
                                      jnp.float32),
        mesh=mesh,
        compiler_params=pltpu.CompilerParams(
            needs_layout_passes=False, use_tc_tiling_on_sc=False),
        scratch_types=[
            pltpu.VMEM((2, 192), jnp.int32),
            pltpu.VMEM((N_PATCHES,), jnp.int32),
            pltpu.VMEM((3, N_PATCHES), jnp.int32),
            pltpu.VMEM((UCELLS // 128, 128), jnp.int32),
            pltpu.VMEM((UCELLS, CELL), jnp.float32),
            pltpu.VMEM((PATCH * C, N_PATCHES), jnp.float32),
            pltpu.SemaphoreType.DMA,
            pltpu.SemaphoreType.DMA,
        ],
    )(_gather_body)
    return k(src, taut, sh, tpat)


def kernel(x_low, x_high, attention, map_index):
    del x_low, map_index
    att_flat = attention.reshape(B, NV)
    u = jax.random.uniform(jax.random.key(42), (B, NV), minval=EPS, maxval=1.0)
    gumbel = -jnp.log(-jnp.log(u))

    samp, taut, sh = _run_sampler(att_flat, gumbel)

    xt = jnp.transpose(x_high.reshape(B, C, HH // 8, 8, WH // 128, 128),
                       (0, 1, 2, 4, 3, 5))
    src = xt.reshape(N_CELLS, CELL)
    out = _run_gather(src, taut, sh, jnp.asarray(_TPAT))
    patches = jnp.transpose(out.reshape(B, PATCH, PATCH, C, N_PATCHES),
                            (0, 4, 1, 2, 3))
    return patches, samp


# speedup vs baseline: 3.9828x; 1.0594x over previous
import functools

import numpy as np
import jax
import jax.numpy as jnp
from jax import lax
from jax.experimental import pallas as pl
from jax.experimental.pallas import tpu as pltpu
from jax.experimental.pallas import tpu_sc as plsc

N_PATCHES = 64
PATCH = 32
B = 8
C = 32
HL = WL = 48
HH = WH = 384
EPS = 1e-8
NV = HL * WL

CELL = 16
CPR = WH // CELL
PLANE = HH * CPR
N_CELLS = B * C * PLANE
PATCH_F32 = PATCH * PATCH * C

NC, NS = 2, 16
NW = NC * NS
CQ = 4
UCELLS = N_PATCHES * (C // CQ) * 3

_mm = np.arange(192)
_TPAT = np.stack([_mm % 3, _mm // 3]).astype(np.int32)


def _sampler_body(att_ref, gum_ref, samp_ref, taut_ref, sh_ref):
    att = att_ref[...]
    scores = jnp.log(att + EPS) + gum_ref[...]
    pos = lax.broadcasted_iota(jnp.int32, (B, NV), 1)
    lane64 = lax.broadcasted_iota(jnp.int32, (B, N_PATCHES), 1)

    def step(i, carry):
        sc, fi, sa = carry
        m = jnp.max(sc, axis=1, keepdims=True)
        idx = jnp.min(jnp.where(sc == m, pos, NV), axis=1, keepdims=True)
        hit = pos == idx
        val = jnp.sum(jnp.where(hit, att, 0.0), axis=1, keepdims=True)
        onehot = lane64 == i
        fi = jnp.where(onehot, idx, fi)
        sa = jnp.where(onehot, val, sa)
        return jnp.where(hit, -jnp.inf, sc), fi, sa

    fi0 = jnp.zeros((B, N_PATCHES), jnp.int32)
    sa0 = jnp.zeros((B, N_PATCHES), jnp.float32)
    _, fi, sa = lax.fori_loop(0, N_PATCHES, step, (scores, fi0, sa0))
    samp_ref[...] = sa

    ys = fi // WL
    xs = fi - ys * WL
    top = jnp.clip(8 * ys - 12, 0, HH - PATCH)
    left = jnp.clip(8 * xs - 12, 0, WH - PATCH)
    sh_ref[...] = left % CELL
    k = lax.broadcasted_iota(jnp.int32, (B, PATCH * 3, N_PATCHES), 1)
    topB = jnp.broadcast_to(top[:, None, :], (B, PATCH * 3, N_PATCHES))
    leftB = jnp.broadcast_to(left[:, None, :], (B, PATCH * 3, N_PATCHES))
    hh = topB + k // 3
    wc16 = jnp.minimum(leftB // CELL + k % 3, CPR - 1)
    taut_ref[...] = (((hh // 8) * 3 + wc16 // 8) * 8 + hh % 8) * 8 + wc16 % 8


def _run_sampler(att_flat, gumbel):
    return pl.pallas_call(
        _sampler_body,
        out_shape=(
            jax.ShapeDtypeStruct((B, N_PATCHES), jnp.float32),
            jax.ShapeDtypeStruct((B, PATCH * 3, N_PATCHES), jnp.int32),
            jax.ShapeDtypeStruct((B, N_PATCHES), jnp.int32),
        ),
    )(att_flat, gumbel)


def _gather_body(src_hbm, taut_hbm, sh_hbm, tpat_hbm, out_hbm,
                 tpat_v, sh_v, tau3_v, idx_v, in_v, out_v, gsem, osem):
    wid = lax.axis_index("s") * NC + lax.axis_index("c")
    b = wid // 4
    ig = wid - b * 4
    bbase = b * (C * PLANE)
    pltpu.sync_copy(tpat_hbm, tpat_v)
    pltpu.sync_copy(sh_hbm.at[b], sh_v)
    lane = lax.iota(jnp.int32, 16)

    def do_row(iu, _):
        i = ig * 8 + iu
        pltpu.sync_copy(taut_hbm.at[b, pl.ds(i * 3, 3)], tau3_v)

        @pl.when(iu >= 1)
        def _():
            pltpu.make_async_copy(
                out_hbm.at[0, 0, :, :, :, pl.ds(0, N_PATCHES)],
                out_v, osem).wait()

        for cq in range(CQ):

            @plsc.parallel_loop(0, UCELLS // 16, unroll=8)
            def build(kk):
                cl = kk // 12
                mmc = (kk - cl * 12) * 16
                tvals = plsc.load_gather(
                    tau3_v, [tpat_v[0, pl.ds(mmc, 16)],
                             tpat_v[1, pl.ds(mmc, 16)]])
                vals = tvals + (bbase + (cq * 8 + cl) * PLANE)
                idx_v[kk // 8, pl.ds((kk % 8) * 16, 16)] = vals

            for kk in range(UCELLS // 128):
                pltpu.async_copy(src_hbm.at[idx_v.at[kk]],
                                 in_v.at[pl.ds(kk * 128, 128)], gsem)
            pltpu.make_async_copy(src_hbm.at[pl.ds(0, UCELLS)], in_v,
                                  gsem).wait()

            for g in range(4):
                rowg = lane * 3 + g * 48
                colg = sh_v[pl.ds(g * 16, 16)]

                @plsc.parallel_loop(0, 256, unroll=8)
                def permute(cj):
                    cl = cj // PATCH
                    j = cj - cl * PATCH
                    v = plsc.load_gather(in_v, [rowg + cl * 192, colg + j])
                    out_v[j, cq, cl, pl.ds(g * 16, 16)] = v

        pltpu.async_copy(out_v,
                         out_hbm.at[b, i, :, :, :, pl.ds(0, N_PATCHES)], osem)
        return 0

    lax.fori_loop(0, 8, do_row, 0)
    pltpu.make_async_copy(out_hbm.at[0, 0, :, :, :, pl.ds(0, N_PATCHES)],
                          out_v, osem).wait()


def _run_gather(src, taut, sh, tpat):
    mesh = plsc.VectorSubcoreMesh(core_axis_name="c", subcore_axis_name="s")
    k = functools.partial(
        pl.kernel,
        out_type=jax.ShapeDtypeStruct((B, PATCH, PATCH, C // 8, 8, 128),
                                      jnp.float32),
        mesh=mesh,
        compiler_params=pltpu.CompilerParams(
            needs_layout_passes=False, use_tc_tiling_on_sc=False),
        scratch_types=[
            pltpu.VMEM((2, 192), jnp.int32),
            pltpu.VMEM((N_PATCHES,), jnp.int32),
            pltpu.VMEM((3, N_PATCHES), jnp.int32),
            pltpu.VMEM((UCELLS // 128, 128), jnp.int32),
            pltpu.VMEM((UCELLS, CELL), jnp.float32),
            pltpu.VMEM((PATCH, C // 8, 8, N_PATCHES), jnp.float32),
            pltpu.SemaphoreType.DMA,
            pltpu.SemaphoreType.DMA,
        ],
    )(_gather_body)
    return k(src, taut, sh, tpat)


def kernel(x_low, x_high, attention, map_index):
    del x_low, map_index
    att_flat = attention.reshape(B, NV)
    u = jax.random.uniform(jax.random.key(42), (B, NV), minval=EPS, maxval=1.0)
    gumbel = -jnp.log(-jnp.log(u))

    samp, taut, sh = _run_sampler(att_flat, gumbel)

    xt = jnp.transpose(x_high.reshape(B, C, HH // 8, 8, WH // 128, 128),
                       (0, 1, 2, 4, 3, 5))
    src = xt.reshape(N_CELLS, CELL)
    out = _run_gather(src, taut, sh, jnp.asarray(_TPAT))
    patches = jnp.transpose(out[:, :, :, :, :, :N_PATCHES],
                            (0, 5, 1, 2, 3, 4)).reshape(
        B, N_PATCHES, PATCH, PATCH, C)
    return patches, samp


# speedup vs baseline: 4.7014x; 1.1804x over previous
import functools

import numpy as np
import jax
import jax.numpy as jnp
from jax import lax
from jax.experimental import pallas as pl
from jax.experimental.pallas import tpu as pltpu
from jax.experimental.pallas import tpu_sc as plsc

N_PATCHES = 64
PATCH = 32
B = 8
C = 32
HL = WL = 48
HH = WH = 384
EPS = 1e-8
NV = HL * WL

CELL = 16
CPR = WH // CELL
PLANE = HH * CPR
N_CELLS = B * C * PLANE
PATCH_F32 = PATCH * PATCH * C

NC, NS = 2, 16
NW = NC * NS
CQ = 4
UCELLS = N_PATCHES * (C // CQ) * 3

_mm = np.arange(192)
_TPAT = np.stack([_mm % 3, _mm // 3]).astype(np.int32)


def _sampler_body(att_ref, gum_ref, samp_ref, taut_ref, sh_ref):
    att = att_ref[...]
    scores = jnp.log(att + EPS) + gum_ref[...]
    pos = lax.broadcasted_iota(jnp.int32, (B, NV), 1)
    lane64 = lax.broadcasted_iota(jnp.int32, (B, N_PATCHES), 1)

    def step(i, carry):
        sc, fi, sa = carry
        m = jnp.max(sc, axis=1, keepdims=True)
        idx = jnp.min(jnp.where(sc == m, pos, NV), axis=1, keepdims=True)
        hit = pos == idx
        val = jnp.sum(jnp.where(hit, att, 0.0), axis=1, keepdims=True)
        onehot = lane64 == i
        fi = jnp.where(onehot, idx, fi)
        sa = jnp.where(onehot, val, sa)
        return jnp.where(hit, -jnp.inf, sc), fi, sa

    fi0 = jnp.zeros((B, N_PATCHES), jnp.int32)
    sa0 = jnp.zeros((B, N_PATCHES), jnp.float32)
    _, fi, sa = lax.fori_loop(0, N_PATCHES, step, (scores, fi0, sa0))
    samp_ref[...] = sa

    ys = fi // WL
    xs = fi - ys * WL
    top = jnp.clip(8 * ys - 12, 0, HH - PATCH)
    left = jnp.clip(8 * xs - 12, 0, WH - PATCH)
    sh_ref[...] = left % CELL
    k = lax.broadcasted_iota(jnp.int32, (B, PATCH * 3, N_PATCHES), 1)
    topB = jnp.broadcast_to(top[:, None, :], (B, PATCH * 3, N_PATCHES))
    leftB = jnp.broadcast_to(left[:, None, :], (B, PATCH * 3, N_PATCHES))
    hh = topB + k // 3
    wc16 = jnp.minimum(leftB // CELL + k % 3, CPR - 1)
    taut_ref[...] = (((hh // 8) * 3 + wc16 // 8) * 8 + hh % 8) * 8 + wc16 % 8


def _run_sampler(att_flat, gumbel):
    return pl.pallas_call(
        _sampler_body,
        out_shape=(
            jax.ShapeDtypeStruct((B, N_PATCHES), jnp.float32),
            jax.ShapeDtypeStruct((B, PATCH * 3, N_PATCHES), jnp.int32),
            jax.ShapeDtypeStruct((B, N_PATCHES), jnp.int32),
        ),
    )(att_flat, gumbel)


def _gather_body(src_hbm, taut_hbm, sh_hbm, tpat_hbm, out_hbm,
                 tpat_v, sh_v, tau3_v, idx_v, in_v, out_v, gsem, osem):
    wid = lax.axis_index("s") * NC + lax.axis_index("c")
    b = wid // 4
    ig = wid - b * 4
    bbase = b * (C * PLANE)
    pltpu.sync_copy(tpat_hbm, tpat_v)
    pltpu.sync_copy(sh_hbm.at[b], sh_v)
    lane = lax.iota(jnp.int32, 16)

    def do_row(iu, _):
        i = ig * 8 + iu
        pltpu.sync_copy(taut_hbm.at[b, pl.ds(i * 3, 3)], tau3_v)

        @pl.when(iu >= 1)
        def _():
            pltpu.make_async_copy(
                out_hbm.at[0, 0, :, :, :, pl.ds(0, N_PATCHES)],
                out_v, osem).wait()

        def build_fire(cq, sl):
            @plsc.parallel_loop(0, UCELLS // 16, unroll=8)
            def build(kk):
                cl = kk // 12
                mmc = (kk - cl * 12) * 16
                tvals = plsc.load_gather(
                    tau3_v, [tpat_v[0, pl.ds(mmc, 16)],
                             tpat_v[1, pl.ds(mmc, 16)]])
                vals = tvals + (bbase + (cq * 8 + cl) * PLANE)
                idx_v[sl, kk // 8, pl.ds((kk % 8) * 16, 16)] = vals

            for kk in range(UCELLS // 128):
                pltpu.async_copy(src_hbm.at[idx_v.at[sl, kk]],
                                 in_v.at[pl.ds(sl * UCELLS + kk * 128, 128)],
                                 gsem.at[sl])

        build_fire(0, 0)
        for cq in range(CQ):
            sl = cq % 2
            if cq < CQ - 1:
                build_fire(cq + 1, (cq + 1) % 2)
            pltpu.make_async_copy(src_hbm.at[pl.ds(0, UCELLS)],
                                  in_v.at[pl.ds(sl * UCELLS, UCELLS)],
                                  gsem.at[sl]).wait()

            for g in range(4):
                rowg = lane * 3 + (g * 48 + sl * UCELLS)
                colg = sh_v[pl.ds(g * 16, 16)]

                @plsc.parallel_loop(0, 256, unroll=8)
                def permute(cj):
                    cl = cj // PATCH
                    j = cj - cl * PATCH
                    v = plsc.load_gather(in_v, [rowg + cl * 192, colg + j])
                    out_v[j, cq, cl, pl.ds(g * 16, 16)] = v

        pltpu.async_copy(out_v,
                         out_hbm.at[b, i, :, :, :, pl.ds(0, N_PATCHES)], osem)
        return 0

    lax.fori_loop(0, 8, do_row, 0)
    pltpu.make_async_copy(out_hbm.at[0, 0, :, :, :, pl.ds(0, N_PATCHES)],
                          out_v, osem).wait()


def _run_gather(src, taut, sh, tpat):
    mesh = plsc.VectorSubcoreMesh(core_axis_name="c", subcore_axis_name="s")
    k = functools.partial(
        pl.kernel,
        out_type=jax.ShapeDtypeStruct((B, PATCH, PATCH, C // 8, 8, 128),
                                      jnp.float32),
        mesh=mesh,
        compiler_params=pltpu.CompilerParams(
            needs_layout_passes=False, use_tc_tiling_on_sc=False),
        scratch_types=[
            pltpu.VMEM((2, 192), jnp.int32),
            pltpu.VMEM((N_PATCHES,), jnp.int32),
            pltpu.VMEM((3, N_PATCHES), jnp.int32),
            pltpu.VMEM((2, UCELLS // 128, 128), jnp.int32),
            pltpu.VMEM((2 * UCELLS, CELL), jnp.float32),
            pltpu.VMEM((PATCH, C // 8, 8, N_PATCHES), jnp.float32),
            pltpu.SemaphoreType.DMA((2,)),
            pltpu.SemaphoreType.DMA,
        ],
    )(_gather_body)
    return k(src, taut, sh, tpat)


def kernel(x_low, x_high, attention, map_index):
    del x_low, map_index
    att_flat = attention.reshape(B, NV)
    u = jax.random.uniform(jax.random.key(42), (B, NV), minval=EPS, maxval=1.0)
    gumbel = -jnp.log(-jnp.log(u))

    samp, taut, sh = _run_sampler(att_flat, gumbel)

    xt = jnp.transpose(x_high.reshape(B, C, HH // 8, 8, WH // 128, 128),
                       (0, 1, 2, 4, 3, 5))
    src = xt.reshape(N_CELLS, CELL)
    out = _run_gather(src, taut, sh, jnp.asarray(_TPAT))
    patches = jnp.transpose(out[:, :, :, :, :, :N_PATCHES],
                            (0, 5, 1, 2, 3, 4)).reshape(
        B, N_PATCHES, PATCH, PATCH, C)
    return patches, samp


# speedup vs baseline: 4.7046x; 1.0007x over previous
import functools

import numpy as np
import jax
import jax.numpy as jnp
from jax import lax
from jax.experimental import pallas as pl
from jax.experimental.pallas import tpu as pltpu
from jax.experimental.pallas import tpu_sc as plsc

N_PATCHES = 64
PATCH = 32
B = 8
C = 32
HL = WL = 48
HH = WH = 384
EPS = 1e-8
NV = HL * WL

CELL = 16
CPR = WH // CELL
PLANE = HH * CPR
N_CELLS = B * C * PLANE

NC = 2
CQ = 4
UCELLS = N_PATCHES * (C // CQ) * 3

_mm = np.arange(192)
_TPAT = np.stack([_mm % 3, _mm // 3]).astype(np.int32)


def _sampler_body(att_ref, gum_ref, samp_ref, taut_ref, sh_ref):
    att = att_ref[...]
    scores = jnp.log(att + EPS) + gum_ref[...]
    pos = lax.broadcasted_iota(jnp.int32, (B, NV), 1)
    lane64 = lax.broadcasted_iota(jnp.int32, (B, N_PATCHES), 1)

    def step(i, carry):
        sc, fi, sa = carry
        m = jnp.max(sc, axis=1, keepdims=True)
        idx = jnp.min(jnp.where(sc == m, pos, NV), axis=1, keepdims=True)
        hit = pos == idx
        val = jnp.sum(jnp.where(hit, att, 0.0), axis=1, keepdims=True)
        onehot = lane64 == i
        fi = jnp.where(onehot, idx, fi)
        sa = jnp.where(onehot, val, sa)
        return jnp.where(hit, -jnp.inf, sc), fi, sa

    fi0 = jnp.zeros((B, N_PATCHES), jnp.int32)
    sa0 = jnp.zeros((B, N_PATCHES), jnp.float32)
    _, fi, sa = lax.fori_loop(0, N_PATCHES, step, (scores, fi0, sa0))
    samp_ref[...] = sa

    ys = fi // WL
    xs = fi - ys * WL
    top = jnp.clip(8 * ys - 12, 0, HH - PATCH)
    left = jnp.clip(8 * xs - 12, 0, WH - PATCH)
    sh_ref[...] = left % CELL
    k = lax.broadcasted_iota(jnp.int32, (B, PATCH * 3, N_PATCHES), 1)
    topB = jnp.broadcast_to(top[:, None, :], (B, PATCH * 3, N_PATCHES))
    leftB = jnp.broadcast_to(left[:, None, :], (B, PATCH * 3, N_PATCHES))
    hh = topB + k // 3
    wc16 = jnp.minimum(leftB // CELL + k % 3, CPR - 1)
    taut_ref[...] = (((hh // 8) * 3 + wc16 // 8) * 8 + hh % 8) * 8 + wc16 % 8


def _run_sampler(att_flat, gumbel):
    return pl.pallas_call(
        _sampler_body,
        out_shape=(
            jax.ShapeDtypeStruct((B, N_PATCHES), jnp.float32),
            jax.ShapeDtypeStruct((B, PATCH * 3, N_PATCHES), jnp.int32),
            jax.ShapeDtypeStruct((B, N_PATCHES), jnp.int32),
        ),
    )(att_flat, gumbel)


def _gather_body(src_hbm, taut_hbm, sh_hbm, tpat_hbm, out_hbm,
                 tpat_v, sh_v, tau3_v, idx_v, in_v, out_v, gsem, osem):
    wid = lax.axis_index("s") * NC + lax.axis_index("c")
    b = wid // 4
    ig = wid - b * 4
    bbase = b * (C * PLANE)
    pltpu.sync_copy(tpat_hbm, tpat_v)
    pltpu.sync_copy(sh_hbm.at[b], sh_v)
    lane = lax.iota(jnp.int32, 16)

    def do_row(iu, _):
        i = ig * 8 + iu
        pltpu.sync_copy(taut_hbm.at[b, pl.ds(i * 3, 3)], tau3_v)

        @pl.when(iu >= 1)
        def _():
            pltpu.make_async_copy(
                out_hbm.at[0, 0, :, :, :, pl.ds(0, N_PATCHES)],
                out_v, osem).wait()

        def build_fire(cq, sl):
            @plsc.parallel_loop(0, UCELLS // 16, unroll=8)
            def build(kk):
                cl = kk // 12
                mmc = (kk - cl * 12) * 16
                tvals = plsc.load_gather(
                    tau3_v, [tpat_v[0, pl.ds(mmc, 16)],
                             tpat_v[1, pl.ds(mmc, 16)]])
                vals = tvals + (bbase + (cq * 8 + cl) * PLANE)
                idx_v[sl, kk // 8, pl.ds((kk % 8) * 16, 16)] = vals

            for kk in range(UCELLS // 128):
                pltpu.async_copy(src_hbm.at[idx_v.at[sl, kk]],
                                 in_v.at[pl.ds(sl * UCELLS + kk * 128, 128)],
                                 gsem.at[sl])

        build_fire(0, 0)
        for cq in range(CQ):
            sl = cq % 2
            if cq < CQ - 1:
                build_fire(cq + 1, (cq + 1) % 2)
            pltpu.make_async_copy(src_hbm.at[pl.ds(0, UCELLS)],
                                  in_v.at[pl.ds(sl * UCELLS, UCELLS)],
                                  gsem.at[sl]).wait()

            for g in range(4):
                rowg = lane * 3 + (g * 48 + sl * UCELLS)
                colg = sh_v[pl.ds(g * 16, 16)]

                @plsc.parallel_loop(0, 256, unroll=8)
                def permute(cj):
                    cl = cj // PATCH
                    j = cj - cl * PATCH
                    v = plsc.load_gather(in_v, [rowg + cl * 192, colg + j])
                    out_v[j, cq, cl, pl.ds(g * 16, 16)] = v

        pltpu.async_copy(out_v,
                         out_hbm.at[b, i, :, :, :, pl.ds(0, N_PATCHES)], osem)
        return 0

    lax.fori_loop(0, 8, do_row, 0)
    pltpu.make_async_copy(out_hbm.at[0, 0, :, :, :, pl.ds(0, N_PATCHES)],
                          out_v, osem).wait()


def _run_gather(src, taut, sh, tpat):
    mesh = plsc.VectorSubcoreMesh(core_axis_name="c", subcore_axis_name="s")
    k = functools.partial(
        pl.kernel,
        out_type=jax.ShapeDtypeStruct((B, PATCH, PATCH, C // 8, 8, 128),
                                      jnp.float32),
        mesh=mesh,
        compiler_params=pltpu.CompilerParams(
            needs_layout_passes=False, use_tc_tiling_on_sc=False),
        scratch_types=[
            pltpu.VMEM((2, 192), jnp.int32),
            pltpu.VMEM((N_PATCHES,), jnp.int32),
            pltpu.VMEM((3, N_PATCHES), jnp.int32),
            pltpu.VMEM((2, UCELLS // 128, 128), jnp.int32),
            pltpu.VMEM((2 * UCELLS, CELL), jnp.float32),
            pltpu.VMEM((PATCH, C // 8, 8, N_PATCHES), jnp.float32),
            pltpu.SemaphoreType.DMA((2,)),
            pltpu.SemaphoreType.DMA,
        ],
    )(_gather_body)
    return k(src, taut, sh, tpat)


def kernel(x_low, x_high, attention, map_index):
    del x_low, map_index
    att_flat = attention.reshape(B, NV)
    u = jax.random.uniform(jax.random.key(42), (B, NV), minval=EPS, maxval=1.0)
    gumbel = -jnp.log(-jnp.log(u))

    samp, taut, sh = _run_sampler(att_flat, gumbel)

    xt = jnp.transpose(x_high.reshape(B, C, HH // 8, 8, WH // 128, 128),
                       (0, 1, 2, 4, 3, 5))
    src = xt.reshape(N_CELLS, CELL)
    out = _run_gather(src, taut, sh, jnp.asarray(_TPAT))
    patches = jnp.transpose(out[:, :, :, :, :, :N_PATCHES],
                            (0, 5, 1, 2, 3, 4)).reshape(
        B, N_PATCHES, PATCH, PATCH, C)
    return patches, samp


# speedup vs baseline: 4.7905x; 1.0183x over previous
import functools

import numpy as np
import jax
import jax.numpy as jnp
from jax import lax
from jax.experimental import pallas as pl
from jax.experimental.pallas import tpu as pltpu
from jax.experimental.pallas import tpu_sc as plsc

N_PATCHES = 64
PATCH = 32
B = 8
C = 32
HL = WL = 48
HH = WH = 384
EPS = 1e-8
NV = HL * WL

CELL = 16
CPR = WH // CELL
PLANE = HH * CPR
N_CELLS = B * C * PLANE

NC = 2
CQ = 4
UCELLS = N_PATCHES * (C // CQ) * 3

_mm = np.arange(192)
_TPAT = np.stack([_mm % 3, _mm // 3]).astype(np.int32)


def _sampler_body(att_ref, gum_ref, samp_ref, taut_ref, sh_ref):
    att = att_ref[...]
    scores = jnp.log(att + EPS) + gum_ref[...]
    pos = lax.broadcasted_iota(jnp.int32, (B, NV), 1)
    lane64 = lax.broadcasted_iota(jnp.int32, (B, N_PATCHES), 1)

    def step(i, carry):
        sc, fi, sa = carry
        m = jnp.max(sc, axis=1, keepdims=True)
        idx = jnp.min(jnp.where(sc == m, pos, NV), axis=1, keepdims=True)
        hit = pos == idx
        val = jnp.sum(jnp.where(hit, att, 0.0), axis=1, keepdims=True)
        onehot = lane64 == i
        fi = jnp.where(onehot, idx, fi)
        sa = jnp.where(onehot, val, sa)
        return jnp.where(hit, -jnp.inf, sc), fi, sa

    fi0 = jnp.zeros((B, N_PATCHES), jnp.int32)
    sa0 = jnp.zeros((B, N_PATCHES), jnp.float32)
    _, fi, sa = lax.fori_loop(0, N_PATCHES, step, (scores, fi0, sa0))
    samp_ref[...] = sa

    ys = fi // WL
    xs = fi - ys * WL
    top = jnp.clip(8 * ys - 12, 0, HH - PATCH)
    left = jnp.clip(8 * xs - 12, 0, WH - PATCH)
    sh_ref[...] = left % CELL
    k = lax.broadcasted_iota(jnp.int32, (B, PATCH * 3, N_PATCHES), 1)
    topB = jnp.broadcast_to(top[:, None, :], (B, PATCH * 3, N_PATCHES))
    leftB = jnp.broadcast_to(left[:, None, :], (B, PATCH * 3, N_PATCHES))
    hh = topB + k // 3
    wc16 = jnp.minimum(leftB // CELL + k % 3, CPR - 1)
    taut_ref[...] = (((hh // 8) * 3 + wc16 // 8) * 8 + hh % 8) * 8 + wc16 % 8


def _run_sampler(att_flat, gumbel):
    return pl.pallas_call(
        _sampler_body,
        out_shape=(
            jax.ShapeDtypeStruct((B, N_PATCHES), jnp.float32),
            jax.ShapeDtypeStruct((B, PATCH * 3, N_PATCHES), jnp.int32),
            jax.ShapeDtypeStruct((B, N_PATCHES), jnp.int32),
        ),
    )(att_flat, gumbel)


def _gather_body(src_hbm, taut_hbm, sh_hbm, tpat_hbm, out_hbm,
                 tpat_v, sh_v, tau3_v, idx_v, in_v, out_v, gsem, osem):
    wid = lax.axis_index("s") * NC + lax.axis_index("c")
    b = wid // 4
    ig = wid - b * 4
    bbase = b * (C * PLANE)
    pltpu.sync_copy(tpat_hbm, tpat_v)
    pltpu.sync_copy(sh_hbm.at[b], sh_v)
    lane = lax.iota(jnp.int32, 16)

    def do_row(iu, _):
        i = ig * 8 + iu
        pltpu.sync_copy(taut_hbm.at[b, pl.ds(i * 3, 3)], tau3_v)

        @pl.when(iu >= 1)
        def _():
            pltpu.make_async_copy(
                out_hbm.at[0, 0, :, :, :, pl.ds(0, N_PATCHES)],
                out_v, osem).wait()

        def build_fire(cq, sl):
            @plsc.parallel_loop(0, UCELLS // 16, unroll=8)
            def build(kk):
                cl = kk // 12
                mmc = (kk - cl * 12) * 16
                tvals = plsc.load_gather(
                    tau3_v, [tpat_v[0, pl.ds(mmc, 16)],
                             tpat_v[1, pl.ds(mmc, 16)]])
                vals = tvals + (bbase + (cq * 8 + cl) * PLANE)
                idx_v[sl, kk // 8, pl.ds((kk % 8) * 16, 16)] = vals

            for kk in range(UCELLS // 128):
                pltpu.async_copy(src_hbm.at[idx_v.at[sl, kk]],
                                 in_v.at[pl.ds(sl * UCELLS + kk * 128, 128)],
                                 gsem.at[sl])

        build_fire(0, 0)
        for cq in range(CQ):
            sl = cq % 2
            if cq < CQ - 1:
                build_fire(cq + 1, (cq + 1) % 2)
            pltpu.make_async_copy(src_hbm.at[pl.ds(0, UCELLS)],
                                  in_v.at[pl.ds(sl * UCELLS, UCELLS)],
                                  gsem.at[sl]).wait()

            for g in range(4):
                rowg = lane * 3 + (g * 48 + sl * UCELLS)
                colg = sh_v[pl.ds(g * 16, 16)]

                @plsc.parallel_loop(0, 256, unroll=16)
                def permute(cj):
                    cl = cj // PATCH
                    j = cj - cl * PATCH
                    v = plsc.load_gather(in_v, [rowg + cl * 192, colg + j])
                    out_v[j, cq, cl, pl.ds(g * 16, 16)] = v

        pltpu.async_copy(out_v,
                         out_hbm.at[b, i, :, :, :, pl.ds(0, N_PATCHES)], osem)
        return 0

    lax.fori_loop(0, 8, do_row, 0)
    pltpu.make_async_copy(out_hbm.at[0, 0, :, :, :, pl.ds(0, N_PATCHES)],
                          out_v, osem).wait()


def _run_gather(src, taut, sh, tpat):
    mesh = plsc.VectorSubcoreMesh(core_axis_name="c", subcore_axis_name="s")
    k = functools.partial(
        pl.kernel,
        out_type=jax.ShapeDtypeStruct((B, PATCH, PATCH, C // 8, 8, 128),
                                      jnp.float32),
        mesh=mesh,
        compiler_params=pltpu.CompilerParams(
            needs_layout_passes=False, use_tc_tiling_on_sc=False),
        scratch_types=[
            pltpu.VMEM((2, 192), jnp.int32),
            pltpu.VMEM((N_PATCHES,), jnp.int32),
            pltpu.VMEM((3, N_PATCHES), jnp.int32),
            pltpu.VMEM((2, UCELLS // 128, 128), jnp.int32),
            pltpu.VMEM((2 * UCELLS, CELL), jnp.float32),
            pltpu.VMEM((PATCH, C // 8, 8, N_PATCHES), jnp.float32),
            pltpu.SemaphoreType.DMA((2,)),
            pltpu.SemaphoreType.DMA,
        ],
    )(_gather_body)
    return k(src, taut, sh, tpat)


def kernel(x_low, x_high, attention, map_index):
    del x_low, map_index
    att_flat = attention.reshape(B, NV)
    u = jax.random.uniform(jax.random.key(42), (B, NV), minval=EPS, maxval=1.0)
    gumbel = -jnp.log(-jnp.log(u))

    samp, taut, sh = _run_sampler(att_flat, gumbel)

    xt = jnp.transpose(x_high.reshape(B, C, HH // 8, 8, WH // 128, 128),
                       (0, 1, 2, 4, 3, 5))
    src = xt.reshape(N_CELLS, CELL)
    out = _run_gather(src, taut, sh, jnp.asarray(_TPAT))
    patches = jnp.transpose(out[:, :, :, :, :, :N_PATCHES],
                            (0, 5, 1, 2, 3, 4)).reshape(
        B, N_PATCHES, PATCH, PATCH, C)
    return patches, samp
